# baseline (speedup 1.0000x reference)
import functools

import jax
import jax.numpy as jnp
from jax import lax
from jax.experimental import pallas as pl
from jax.experimental.pallas import tpu as pltpu
from jax.experimental.pallas import tpu_sc as plsc

D = 64
VOCAB = 1000000
DP = 128
NC = 2
NS = 16
NW = NC * NS
CHUNK = 800


def _make_emb(b_total):
    b_per_w = b_total // NW
    n_pairs = b_per_w // (2 * CHUNK)
    mesh = plsc.VectorSubcoreMesh(core_axis_name="c", subcore_axis_name="s")

    @functools.partial(
        pl.kernel,
        mesh=mesh,
        compiler_params=pltpu.CompilerParams(use_tc_tiling_on_sc=False),
        out_type=jax.ShapeDtypeStruct((b_total, DP), jnp.float32),
        scratch_types=[
            pltpu.VMEM((CHUNK,), jnp.int32),
            pltpu.VMEM((CHUNK,), jnp.int32),
            pltpu.VMEM((CHUNK, D), jnp.float32),
            pltpu.VMEM((CHUNK, D), jnp.float32),
            pltpu.SemaphoreType.DMA,
            pltpu.SemaphoreType.DMA,
            pltpu.SemaphoreType.DMA,
            pltpu.SemaphoreType.DMA,
            pltpu.SemaphoreType.DMA,
        ],
    )
    def emb(idx_hbm, table_hbm, out_hbm, idx0, idx1, rows0, rows1,
            sem_idx, sem_ga, sem_gb, sem_o0, sem_o1):
        wid = lax.axis_index("s") * NC + lax.axis_index("c")
        base = wid * b_per_w

        pltpu.async_copy(idx_hbm.at[pl.ds(base, CHUNK)], idx0, sem_idx)

        def body(k, carry):
            off_a = base + (2 * k) * CHUNK
            off_b = off_a + CHUNK

            pltpu.make_async_copy(
                idx_hbm.at[pl.ds(0, CHUNK)], idx0, sem_idx).wait()
            pltpu.async_copy(idx_hbm.at[pl.ds(off_b, CHUNK)], idx1, sem_idx)

            @pl.when(k > 0)
            def _():
                pltpu.make_async_copy(
                    rows0,
                    out_hbm.at[pl.ds(0, CHUNK), pl.ds(0, D)], sem_o0).wait()

            ga = pltpu.async_copy(table_hbm.at[idx0], rows0, sem_ga)

            pltpu.make_async_copy(
                idx_hbm.at[pl.ds(0, CHUNK)], idx1, sem_idx).wait()

            @pl.when(k > 0)
            def _():
                pltpu.make_async_copy(
                    rows1,
                    out_hbm.at[pl.ds(0, CHUNK), pl.ds(0, D)], sem_o1).wait()

            gb = pltpu.async_copy(table_hbm.at[idx1], rows1, sem_gb)

            ga.wait()
            @pl.when(k + 1 < n_pairs)
            def _():
                pltpu.async_copy(
                    idx_hbm.at[pl.ds(off_b + CHUNK, CHUNK)], idx0, sem_idx)

            pltpu.async_copy(rows0,
                             out_hbm.at[pl.ds(off_a, CHUNK), pl.ds(0, D)],
                             sem_o0)
            gb.wait()
            pltpu.async_copy(rows1,
                             out_hbm.at[pl.ds(off_b, CHUNK), pl.ds(0, D)],
                             sem_o1)
            return carry

        lax.fori_loop(0, n_pairs, body, 0)

        pltpu.make_async_copy(
            rows0,
            out_hbm.at[pl.ds(0, CHUNK), pl.ds(0, D)], sem_o0).wait()
        pltpu.make_async_copy(
            rows1,
            out_hbm.at[pl.ds(0, CHUNK), pl.ds(0, D)], sem_o1).wait()

    return emb


_emb = _make_emb(4096 * 200)

BT = 1024


def _tp_body(x_ref, o_ref):
    eye = jnp.eye(D, dtype=jnp.float32)
    o_ref[:, 0:D] = jax.lax.dot_general(
        x_ref[...], eye, dimension_numbers=(((0,), (0,)), ((), ())),
        preferred_element_type=jnp.float32)


_transpose_pad = pl.pallas_call(
    _tp_body,
    grid=((VOCAB + BT - 1) // BT,),
    in_specs=[pl.BlockSpec((D, BT), lambda i: (0, i))],
    out_specs=pl.BlockSpec((BT, DP), lambda i: (i, 0)),
    out_shape=jax.ShapeDtypeStruct((VOCAB, DP), jnp.float32),
)


@jax.jit
def kernel(x_ids, table):
    flat = x_ids.reshape(-1) * 2
    tbl = _transpose_pad(table.T).reshape(2 * VOCAB, D)
    out = _emb(flat, tbl)
    return out[:, :D].reshape(x_ids.shape + (D,))


# speedup vs baseline: 1.2699x; 1.2699x over previous
import functools

import jax
import jax.numpy as jnp
from jax import lax
from jax.experimental import pallas as pl
from jax.experimental.pallas import tpu as pltpu
from jax.experimental.pallas import tpu_sc as plsc

D = 64
VOCAB = 1000000
DP = 128
NC = 2
NS = 16
NW = NC * NS
CHUNK = 800


def _make_emb(b_total):
    b_per_w = b_total // NW
    n_pairs = b_per_w // (2 * CHUNK)
    mesh = plsc.VectorSubcoreMesh(core_axis_name="c", subcore_axis_name="s")

    @functools.partial(
        pl.kernel,
        mesh=mesh,
        compiler_params=pltpu.CompilerParams(use_tc_tiling_on_sc=False),
        out_type=jax.ShapeDtypeStruct((b_total, DP), jnp.float32),
        scratch_types=[
            pltpu.VMEM((CHUNK,), jnp.int32),
            pltpu.VMEM((CHUNK,), jnp.int32),
            pltpu.VMEM((CHUNK, D), jnp.float32),
            pltpu.VMEM((CHUNK, D), jnp.float32),
            pltpu.SemaphoreType.DMA,
            pltpu.SemaphoreType.DMA,
            pltpu.SemaphoreType.DMA,
            pltpu.SemaphoreType.DMA,
            pltpu.SemaphoreType.DMA,
        ],
    )
    def emb(idx_hbm, table_hbm, out_hbm, idx0, idx1, rows0, rows1,
            sem_idx, sem_ga, sem_gb, sem_o0, sem_o1):
        wid = lax.axis_index("s") * NC + lax.axis_index("c")
        base = wid * b_per_w

        pltpu.async_copy(idx_hbm.at[pl.ds(base, CHUNK)], idx0, sem_idx)

        def body(k, carry):
            off_a = base + (2 * k) * CHUNK
            off_b = off_a + CHUNK

            pltpu.make_async_copy(
                idx_hbm.at[pl.ds(0, CHUNK)], idx0, sem_idx).wait()
            pltpu.async_copy(idx_hbm.at[pl.ds(off_b, CHUNK)], idx1, sem_idx)

            @pl.when(k > 0)
            def _():
                pltpu.make_async_copy(
                    rows0,
                    out_hbm.at[pl.ds(0, CHUNK), pl.ds(0, D)], sem_o0).wait()

            ga = pltpu.async_copy(table_hbm.at[idx0], rows0, sem_ga)

            pltpu.make_async_copy(
                idx_hbm.at[pl.ds(0, CHUNK)], idx1, sem_idx).wait()

            @pl.when(k > 0)
            def _():
                pltpu.make_async_copy(
                    rows1,
                    out_hbm.at[pl.ds(0, CHUNK), pl.ds(0, D)], sem_o1).wait()

            gb = pltpu.async_copy(table_hbm.at[idx1], rows1, sem_gb)

            ga.wait()
            @pl.when(k + 1 < n_pairs)
            def _():
                pltpu.async_copy(
                    idx_hbm.at[pl.ds(off_b + CHUNK, CHUNK)], idx0, sem_idx)

            pltpu.async_copy(rows0,
                             out_hbm.at[pl.ds(off_a, CHUNK), pl.ds(0, D)],
                             sem_o0)
            gb.wait()
            pltpu.async_copy(rows1,
                             out_hbm.at[pl.ds(off_b, CHUNK), pl.ds(0, D)],
                             sem_o1)
            return carry

        lax.fori_loop(0, n_pairs, body, 0)

        pltpu.make_async_copy(
            rows0,
            out_hbm.at[pl.ds(0, CHUNK), pl.ds(0, D)], sem_o0).wait()
        pltpu.make_async_copy(
            rows1,
            out_hbm.at[pl.ds(0, CHUNK), pl.ds(0, D)], sem_o1).wait()

    return emb


_emb = _make_emb(4096 * 200)


@jax.jit
def kernel(x_ids, table):
    flat = x_ids.reshape(-1) * 2
    tbl = jnp.pad(table, ((0, 0), (0, DP - D))).reshape(2 * VOCAB, D)
    out = _emb(flat, tbl)
    return out[:, :D].reshape(x_ids.shape + (D,))
